# baseline (speedup 1.0000x reference)
import functools

import jax
import jax.numpy as jnp
from jax import lax
from jax.experimental import pallas as pl
from jax.experimental.pallas import tpu as pltpu
from jax.experimental.pallas import tpu_sc as plsc

N = 10000
E = 320000
IN = 128
HID = 256
HH = 128
OUT = 128
G = 64

NC = 2
NS = 16
CH = 80
NP = 10240
RPS = NP // NS

R = 400
NB = N // R

@functools.cache
def _sc_degree_fn():
    mesh = plsc.VectorSubcoreMesh(core_axis_name="c", subcore_axis_name="s")
    return functools.partial(
        pl.kernel,
        mesh=mesh,
        out_type=jax.ShapeDtypeStruct((NC, NP, 16), jnp.float32),
        scratch_types=[
            pltpu.VMEM_SHARED((NP, 16), jnp.float32),
            pltpu.VMEM((RPS, 16), jnp.float32),
            pltpu.VMEM((CH, 16), jnp.float32),
            pltpu.VMEM((1, CH), jnp.int32),
        ],
    )(_sc_degree_body)


def _sc_degree_body(dst_hbm, out_hbm, acc, zbuf, obuf, ibuf):
    c = lax.axis_index("c")
    s = lax.axis_index("s")

    def _zrow(i, _):
        zbuf[i, :] = jnp.zeros((16,), jnp.float32)
        return _
    lax.fori_loop(0, RPS, _zrow, None)

    def _orow(i, _):
        obuf[i, :] = jnp.full((16,), 1.0, jnp.float32)
        return _
    lax.fori_loop(0, CH, _orow, None)

    pltpu.sync_copy(zbuf, acc.at[pl.ds(s * RPS, RPS), :])
    plsc.subcore_barrier()

    epw = E // (NC * NS)
    base0 = (c * NS + s) * epw

    def _step(j, _):
        pltpu.sync_copy(dst_hbm.at[pl.ds(base0 + j * CH, CH)], ibuf.at[0])
        pltpu.sync_copy(obuf, acc.at[ibuf.at[0]], add=True)
        return _
    lax.fori_loop(0, epw // CH, _step, None)

    plsc.subcore_barrier()
    pltpu.sync_copy(acc.at[pl.ds(s * RPS, RPS), :],
                    out_hbm.at[c, pl.ds(s * RPS, RPS), :])


EPW = E // NS
NCH = EPW // CH
DR = 4
DI = 8
NST = 256


@functools.cache
def _sc_aggregate_fn():
    mesh = plsc.VectorSubcoreMesh(core_axis_name="c", subcore_axis_name="s")
    return functools.partial(
        pl.kernel,
        mesh=mesh,
        out_type=jax.ShapeDtypeStruct((NC, NP, HH), jnp.float32),
        scratch_types=(
            [pltpu.VMEM_SHARED((NP, HH), jnp.float32)]
            + [pltpu.VMEM((CH, HH), jnp.float32) for _ in range(DR)]
            + [pltpu.VMEM((DI, CH), jnp.int32),
               pltpu.VMEM((DI, CH), jnp.int32)]
            + [pltpu.SemaphoreType.DMA for _ in range(1 + 2 * DR + DI)]
        ),
    )(_sc_aggregate_body)


def _sc_aggregate_body(src_hbm, dst_hbm, y_hbm, out_hbm, acc, *sc):
    rows = sc[:DR]
    sidx, didx = sc[DR], sc[DR + 1]
    isem = sc[DR + 2]
    gsem = sc[DR + 3:DR + 3 + DR]
    ssem = sc[DR + 3 + DR:DR + 3 + 2 * DR]
    xsem = sc[DR + 3 + 2 * DR:]
    c = lax.axis_index("c")
    s = lax.axis_index("s")
    e0 = s * EPW
    coff = c * NP

    init = pltpu.make_async_copy(y_hbm.at[pl.ds(c * NP + s * RPS, RPS), :],
                                 acc.at[pl.ds(s * RPS, RPS), :], isem)
    init.start()

    def _idx_copies(j, q):
        base = e0 + j * CH
        return (pltpu.make_async_copy(src_hbm.at[pl.ds(base, CH)],
                                      sidx.at[q], xsem[q]),
                pltpu.make_async_copy(dst_hbm.at[pl.ds(base, CH)],
                                      didx.at[q], xsem[q]))

    def _fire_idx(j, q):
        for cp in _idx_copies(j, q):
            cp.start()

    def _gather(b, q):
        return pltpu.make_async_copy(y_hbm.at[sidx.at[q]], rows[b], gsem[b])

    def _prep(j, b, q):
        for cp in _idx_copies(j, q):
            cp.wait()
        for k in range(CH // 16):
            sidx[q, pl.ds(k * 16, 16)] = sidx[q, pl.ds(k * 16, 16)] + coff
        _gather(b, q).start()

    def _scatter(b, q):
        return pltpu.make_async_copy(rows[b], acc.at[didx.at[q]], ssem[b])

    for q0 in range(5):
        _fire_idx(q0, q0)
    _prep(0, 0, 0)
    init.wait()
    plsc.subcore_barrier()

    def _outer(t, _):
        for u in range(DI):
            j = t * DI + u
            b, q = u % DR, u

            @pl.when(j < NCH)
            def _gw():
                _gather(b, q).wait()
                _scatter(b, q).start(add=True)

            @pl.when((j >= 2) & (j < NCH + 2))
            def _sw():
                _scatter((u + 2) % DR, (u + 6) % DI).wait()

            @pl.when(j + 1 < NCH)
            def _pp():
                _prep(j + 1, (u + 1) % DR, (u + 1) % DI)

            @pl.when(j + 5 < NCH)
            def _fi():
                _fire_idx(j + 5, (u + 5) % DI)
        return _
    lax.fori_loop(0, NST // DI, _outer, None)

    plsc.subcore_barrier()
    pltpu.sync_copy(acc.at[pl.ds(s * RPS, RPS), :],
                    out_hbm.at[c, pl.ds(s * RPS, RPS), :])


def _dinv_of(dp):
    deg = dp[0, :, 0] + dp[1, :, 0] + 1.0
    return lax.rsqrt(deg)[:, None]


def _tc_prologue_body(x_ref, w_ref, dp_ref, y_ref):
    dinv = _dinv_of(dp_ref[...])
    xw = jnp.dot(x_ref[...], w_ref[...], preferred_element_type=jnp.float32)
    y = xw * dinv
    y_ref[0] = y[:, :HH]
    y_ref[1] = y[:, HH:]


def _tc_prologue(x, W1, dp):
    return pl.pallas_call(
        _tc_prologue_body,
        grid=(NB,),
        in_specs=[
            pl.BlockSpec((R, IN), lambda i: (i, 0)),
            pl.BlockSpec((IN, HID), lambda i: (0, 0)),
            pl.BlockSpec((2, R, 16), lambda i: (0, i, 0)),
        ],
        out_specs=pl.BlockSpec((2, R, HH), lambda i: (0, i, 0)),
        out_shape=jax.ShapeDtypeStruct((2, NP, HH), jnp.float32),
    )(x, W1, dp)


def _tc_mid_body(a_ref, dp_ref, b_ref, w_ref, y_ref):
    dinv = _dinv_of(dp_ref[...])
    agg = jnp.concatenate([a_ref[0], a_ref[1]], axis=1)
    h = jax.nn.relu(agg * dinv + b_ref[...])
    hw = jnp.dot(h, w_ref[...], preferred_element_type=jnp.float32)
    y = hw * dinv
    y_ref[0] = y[:, :HH]
    y_ref[1] = y[:, HH:]


def _tc_mid(agg1, dp, b1, W2):
    return pl.pallas_call(
        _tc_mid_body,
        grid=(NB,),
        in_specs=[
            pl.BlockSpec((2, R, HH), lambda i: (0, i, 0)),
            pl.BlockSpec((2, R, 16), lambda i: (0, i, 0)),
            pl.BlockSpec((1, HID), lambda i: (0, 0)),
            pl.BlockSpec((HID, HID), lambda i: (0, 0)),
        ],
        out_specs=pl.BlockSpec((2, R, HH), lambda i: (0, i, 0)),
        out_shape=jax.ShapeDtypeStruct((2, NP, HH), jnp.float32),
    )(agg1, dp, b1, W2)


def _tc_final_body(a_ref, dp_ref, b_ref, batch_ref, wl_ref, bl_ref, o_ref,
                   pool_acc, cnt_acc):
    i = pl.program_id(0)
    dinv = _dinv_of(dp_ref[...])
    agg = jnp.concatenate([a_ref[0], a_ref[1]], axis=1)
    h = jax.nn.relu(agg * dinv + b_ref[...])

    bv = batch_ref[0, 0, :]
    sel = (lax.broadcasted_iota(jnp.int32, (G, R), 0) == bv[None, :])
    S = sel.astype(jnp.float32)

    @pl.when(i == 0)
    def _init():
        pool_acc[...] = jnp.zeros((G, HID), jnp.float32)
        cnt_acc[...] = jnp.zeros((G, 128), jnp.float32)

    pool_acc[...] += jnp.dot(S, h, preferred_element_type=jnp.float32)
    cnt_acc[...] += jnp.broadcast_to(jnp.sum(S, axis=1, keepdims=True),
                                     (G, 128))

    @pl.when(i == NB - 1)
    def _fin():
        cnt = cnt_acc[...][:, 0:1]
        pooled = pool_acc[...] / jnp.maximum(cnt, 1.0)
        o_ref[...] = (jnp.dot(pooled, wl_ref[...],
                              preferred_element_type=jnp.float32)
                      + bl_ref[...])


def _tc_final(agg2, dp, b2, batch3, Wl, bl):
    return pl.pallas_call(
        _tc_final_body,
        grid=(NB,),
        in_specs=[
            pl.BlockSpec((2, R, HH), lambda i: (0, i, 0)),
            pl.BlockSpec((2, R, 16), lambda i: (0, i, 0)),
            pl.BlockSpec((1, HID), lambda i: (0, 0)),
            pl.BlockSpec((1, 1, R), lambda i: (i, 0, 0)),
            pl.BlockSpec((HID, OUT), lambda i: (0, 0)),
            pl.BlockSpec((1, OUT), lambda i: (0, 0)),
        ],
        out_specs=pl.BlockSpec((G, OUT), lambda i: (0, 0)),
        out_shape=jax.ShapeDtypeStruct((G, OUT), jnp.float32),
        scratch_shapes=[
            pltpu.VMEM((G, HID), jnp.float32),
            pltpu.VMEM((G, 128), jnp.float32),
        ],
    )(agg2, dp, b2, batch3, Wl, bl)


def kernel(x, edge_index, batch, W1, b1, W2, b2, Wl, bl):
    src = edge_index[0]
    dst = edge_index[1]
    dp = _sc_degree_fn()(dst)
    y1 = _tc_prologue(x, W1, dp)
    agg1 = _sc_aggregate_fn()(src, dst, y1.reshape(2 * NP, HH))
    y2 = _tc_mid(agg1, dp, b1.reshape(1, HID), W2)
    agg2 = _sc_aggregate_fn()(src, dst, y2.reshape(2 * NP, HH))
    return _tc_final(agg2, dp, b2.reshape(1, HID),
                     batch.reshape(NB, 1, R), Wl, bl.reshape(1, OUT))


# speedup vs baseline: 1.4420x; 1.4420x over previous
import functools

import jax
import jax.numpy as jnp
from jax import lax
from jax.experimental import pallas as pl
from jax.experimental.pallas import tpu as pltpu
from jax.experimental.pallas import tpu_sc as plsc

N = 10000
E = 320000
IN = 128
HID = 256
HH = 128
OUT = 128
G = 64

NC = 2
NS = 16
CH = 80
NP = 10240
RPS = NP // NS

R = 400
NB = N // R

@functools.cache
def _sc_degree_fn():
    mesh = plsc.VectorSubcoreMesh(core_axis_name="c", subcore_axis_name="s")
    return functools.partial(
        pl.kernel,
        mesh=mesh,
        out_type=jax.ShapeDtypeStruct((NC, NP, 16), jnp.float32),
        scratch_types=[
            pltpu.VMEM_SHARED((NP, 16), jnp.float32),
            pltpu.VMEM((RPS, 16), jnp.float32),
            pltpu.VMEM((CH, 16), jnp.float32),
            pltpu.VMEM((1, CH), jnp.int32),
        ],
    )(_sc_degree_body)


def _sc_degree_body(dst_hbm, out_hbm, acc, zbuf, obuf, ibuf):
    c = lax.axis_index("c")
    s = lax.axis_index("s")

    def _zrow(i, _):
        zbuf[i, :] = jnp.zeros((16,), jnp.float32)
        return _
    lax.fori_loop(0, RPS, _zrow, None)

    def _orow(i, _):
        obuf[i, :] = jnp.full((16,), 1.0, jnp.float32)
        return _
    lax.fori_loop(0, CH, _orow, None)

    pltpu.sync_copy(zbuf, acc.at[pl.ds(s * RPS, RPS), :])
    plsc.subcore_barrier()

    epw = E // (NC * NS)
    base0 = (c * NS + s) * epw

    def _step(j, _):
        pltpu.sync_copy(dst_hbm.at[pl.ds(base0 + j * CH, CH)], ibuf.at[0])
        pltpu.sync_copy(obuf, acc.at[ibuf.at[0]], add=True)
        return _
    lax.fori_loop(0, epw // CH, _step, None)

    plsc.subcore_barrier()
    pltpu.sync_copy(acc.at[pl.ds(s * RPS, RPS), :],
                    out_hbm.at[c, pl.ds(s * RPS, RPS), :])


EPW = E // NS
NCH = EPW // CH
DR = 3
DI = 6
NST = 252


@functools.cache
def _sc_aggregate_fn():
    mesh = plsc.VectorSubcoreMesh(core_axis_name="c", subcore_axis_name="s")
    return functools.partial(
        pl.kernel,
        mesh=mesh,
        out_type=jax.ShapeDtypeStruct((NC, NP, HH), jnp.float32),
        scratch_types=(
            [pltpu.VMEM_SHARED((NP, HH), jnp.float32)]
            + [pltpu.VMEM((CH, HH), jnp.float32) for _ in range(DR)]
            + [pltpu.VMEM((DI, CH), jnp.int32),
               pltpu.VMEM((DI, CH), jnp.int32)]
            + [pltpu.SemaphoreType.DMA for _ in range(1 + DR + DI)]
        ),
    )(_sc_aggregate_body)


def _sc_aggregate_body(src_hbm, dst_hbm, y_hbm, out_hbm, acc, *sc):
    rows = sc[:DR]
    sidx, didx = sc[DR], sc[DR + 1]
    isem = sc[DR + 2]
    gsem = sc[DR + 3:DR + 3 + DR]
    xsem = sc[DR + 3 + DR:]
    c = lax.axis_index("c")
    s = lax.axis_index("s")
    e0 = s * EPW
    coff = c * NP

    init = pltpu.make_async_copy(y_hbm.at[pl.ds(c * NP + s * RPS, RPS), :],
                                 acc.at[pl.ds(s * RPS, RPS), :], isem)
    init.start()

    def _idx_copies(j, q):
        base = e0 + j * CH
        return (pltpu.make_async_copy(src_hbm.at[pl.ds(base, CH)],
                                      sidx.at[q], xsem[q]),
                pltpu.make_async_copy(dst_hbm.at[pl.ds(base, CH)],
                                      didx.at[q], xsem[q]))

    def _fire_idx(j, q):
        for cp in _idx_copies(j, q):
            cp.start()

    def _gather(b, q):
        return pltpu.make_async_copy(y_hbm.at[sidx.at[q]], rows[b], gsem[b])

    def _prep(j, b, q):
        for cp in _idx_copies(j, q):
            cp.wait()
        for k in range(CH // 16):
            sidx[q, pl.ds(k * 16, 16)] = sidx[q, pl.ds(k * 16, 16)] + coff
        _gather(b, q).start()

    for q0 in range(5):
        _fire_idx(q0, q0)
    _prep(0, 0, 0)
    _prep(1, 1, 1)
    init.wait()
    plsc.subcore_barrier()

    def _outer(t, _):
        for u in range(DI):
            j = t * DI + u
            b, q = u % DR, u

            @pl.when(j < NCH)
            def _gw():
                _gather(b, q).wait()

            @pl.when(j + 2 < NCH)
            def _pp():
                _prep(j + 2, (u + 2) % DR, (u + 2) % DI)

            @pl.when(j < NCH)
            def _sc():
                pltpu.sync_copy(rows[b], acc.at[didx.at[q]], add=True)

            @pl.when(j + 5 < NCH)
            def _fi():
                _fire_idx(j + 5, (u + 5) % DI)
        return _
    lax.fori_loop(0, NST // DI, _outer, None)

    plsc.subcore_barrier()
    pltpu.sync_copy(acc.at[pl.ds(s * RPS, RPS), :],
                    out_hbm.at[c, pl.ds(s * RPS, RPS), :])


def _dinv_of(dp):
    deg = dp[0, :, 0] + dp[1, :, 0] + 1.0
    return lax.rsqrt(deg)[:, None]


def _tc_prologue_body(x_ref, w_ref, dp_ref, y_ref):
    dinv = _dinv_of(dp_ref[...])
    xw = jnp.dot(x_ref[...], w_ref[...], preferred_element_type=jnp.float32)
    y = xw * dinv
    y_ref[0] = y[:, :HH]
    y_ref[1] = y[:, HH:]


def _tc_prologue(x, W1, dp):
    return pl.pallas_call(
        _tc_prologue_body,
        grid=(NB,),
        in_specs=[
            pl.BlockSpec((R, IN), lambda i: (i, 0)),
            pl.BlockSpec((IN, HID), lambda i: (0, 0)),
            pl.BlockSpec((2, R, 16), lambda i: (0, i, 0)),
        ],
        out_specs=pl.BlockSpec((2, R, HH), lambda i: (0, i, 0)),
        out_shape=jax.ShapeDtypeStruct((2, NP, HH), jnp.float32),
    )(x, W1, dp)


def _tc_mid_body(a_ref, dp_ref, b_ref, w_ref, y_ref):
    dinv = _dinv_of(dp_ref[...])
    agg = jnp.concatenate([a_ref[0], a_ref[1]], axis=1)
    h = jax.nn.relu(agg * dinv + b_ref[...])
    hw = jnp.dot(h, w_ref[...], preferred_element_type=jnp.float32)
    y = hw * dinv
    y_ref[0] = y[:, :HH]
    y_ref[1] = y[:, HH:]


def _tc_mid(agg1, dp, b1, W2):
    return pl.pallas_call(
        _tc_mid_body,
        grid=(NB,),
        in_specs=[
            pl.BlockSpec((2, R, HH), lambda i: (0, i, 0)),
            pl.BlockSpec((2, R, 16), lambda i: (0, i, 0)),
            pl.BlockSpec((1, HID), lambda i: (0, 0)),
            pl.BlockSpec((HID, HID), lambda i: (0, 0)),
        ],
        out_specs=pl.BlockSpec((2, R, HH), lambda i: (0, i, 0)),
        out_shape=jax.ShapeDtypeStruct((2, NP, HH), jnp.float32),
    )(agg1, dp, b1, W2)


def _tc_final_body(a_ref, dp_ref, b_ref, batch_ref, wl_ref, bl_ref, o_ref,
                   pool_acc, cnt_acc):
    i = pl.program_id(0)
    dinv = _dinv_of(dp_ref[...])
    agg = jnp.concatenate([a_ref[0], a_ref[1]], axis=1)
    h = jax.nn.relu(agg * dinv + b_ref[...])

    bv = batch_ref[0, 0, :]
    sel = (lax.broadcasted_iota(jnp.int32, (G, R), 0) == bv[None, :])
    S = sel.astype(jnp.float32)

    @pl.when(i == 0)
    def _init():
        pool_acc[...] = jnp.zeros((G, HID), jnp.float32)
        cnt_acc[...] = jnp.zeros((G, 128), jnp.float32)

    pool_acc[...] += jnp.dot(S, h, preferred_element_type=jnp.float32)
    cnt_acc[...] += jnp.broadcast_to(jnp.sum(S, axis=1, keepdims=True),
                                     (G, 128))

    @pl.when(i == NB - 1)
    def _fin():
        cnt = cnt_acc[...][:, 0:1]
        pooled = pool_acc[...] / jnp.maximum(cnt, 1.0)
        o_ref[...] = (jnp.dot(pooled, wl_ref[...],
                              preferred_element_type=jnp.float32)
                      + bl_ref[...])


def _tc_final(agg2, dp, b2, batch3, Wl, bl):
    return pl.pallas_call(
        _tc_final_body,
        grid=(NB,),
        in_specs=[
            pl.BlockSpec((2, R, HH), lambda i: (0, i, 0)),
            pl.BlockSpec((2, R, 16), lambda i: (0, i, 0)),
            pl.BlockSpec((1, HID), lambda i: (0, 0)),
            pl.BlockSpec((1, 1, R), lambda i: (i, 0, 0)),
            pl.BlockSpec((HID, OUT), lambda i: (0, 0)),
            pl.BlockSpec((1, OUT), lambda i: (0, 0)),
        ],
        out_specs=pl.BlockSpec((G, OUT), lambda i: (0, 0)),
        out_shape=jax.ShapeDtypeStruct((G, OUT), jnp.float32),
        scratch_shapes=[
            pltpu.VMEM((G, HID), jnp.float32),
            pltpu.VMEM((G, 128), jnp.float32),
        ],
    )(agg2, dp, b2, batch3, Wl, bl)


def kernel(x, edge_index, batch, W1, b1, W2, b2, Wl, bl):
    src = edge_index[0]
    dst = edge_index[1]
    dp = _sc_degree_fn()(dst)
    y1 = _tc_prologue(x, W1, dp)
    agg1 = _sc_aggregate_fn()(src, dst, y1.reshape(2 * NP, HH))
    y2 = _tc_mid(agg1, dp, b1.reshape(1, HID), W2)
    agg2 = _sc_aggregate_fn()(src, dst, y2.reshape(2 * NP, HH))
    return _tc_final(agg2, dp, b2.reshape(1, HID),
                     batch.reshape(NB, 1, R), Wl, bl.reshape(1, OUT))


# speedup vs baseline: 1.5161x; 1.0514x over previous
import functools

import jax
import jax.numpy as jnp
from jax import lax
from jax.experimental import pallas as pl
from jax.experimental.pallas import tpu as pltpu
from jax.experimental.pallas import tpu_sc as plsc

N = 10000
E = 320000
IN = 128
HID = 256
HH = 128
OUT = 128
G = 64

NC = 2
NS = 16
CH = 80
NP = 10240
RPS = NP // NS

R = 400
NB = N // R

@functools.cache
def _sc_degree_fn():
    mesh = plsc.VectorSubcoreMesh(core_axis_name="c", subcore_axis_name="s")
    return functools.partial(
        pl.kernel,
        mesh=mesh,
        out_type=jax.ShapeDtypeStruct((NC, NP, 16), jnp.float32),
        scratch_types=[
            pltpu.VMEM_SHARED((NP, 16), jnp.float32),
            pltpu.VMEM((RPS, 16), jnp.float32),
            pltpu.VMEM((CH, 16), jnp.float32),
            pltpu.VMEM((1, CH), jnp.int32),
        ],
    )(_sc_degree_body)


def _sc_degree_body(dst_hbm, out_hbm, acc, zbuf, obuf, ibuf):
    c = lax.axis_index("c")
    s = lax.axis_index("s")

    def _zrow(i, _):
        zbuf[i, :] = jnp.zeros((16,), jnp.float32)
        return _
    lax.fori_loop(0, RPS, _zrow, None)

    def _orow(i, _):
        obuf[i, :] = jnp.full((16,), 1.0, jnp.float32)
        return _
    lax.fori_loop(0, CH, _orow, None)

    pltpu.sync_copy(zbuf, acc.at[pl.ds(s * RPS, RPS), :])
    plsc.subcore_barrier()

    epw = E // (NC * NS)
    base0 = (c * NS + s) * epw

    def _step(j, _):
        pltpu.sync_copy(dst_hbm.at[pl.ds(base0 + j * CH, CH)], ibuf.at[0])
        pltpu.sync_copy(obuf, acc.at[ibuf.at[0]], add=True)
        return _
    lax.fori_loop(0, epw // CH, _step, None)

    plsc.subcore_barrier()
    pltpu.sync_copy(acc.at[pl.ds(s * RPS, RPS), :],
                    out_hbm.at[c, pl.ds(s * RPS, RPS), :])


EPW = E // NS
NCH = EPW // CH
DR = 4
DI = 8
NST = 256


@functools.cache
def _sc_aggregate_fn():
    mesh = plsc.VectorSubcoreMesh(core_axis_name="c", subcore_axis_name="s")
    return functools.partial(
        pl.kernel,
        mesh=mesh,
        out_type=jax.ShapeDtypeStruct((NC, NP, HH), jnp.float32),
        scratch_types=(
            [pltpu.VMEM_SHARED((NP, HH), jnp.float32)]
            + [pltpu.VMEM((CH, HH), jnp.float32) for _ in range(DR)]
            + [pltpu.VMEM((DI, CH), jnp.int32),
               pltpu.VMEM((DI, CH), jnp.int32)]
            + [pltpu.SemaphoreType.DMA for _ in range(1 + DR + DI)]
        ),
    )(_sc_aggregate_body)


def _sc_aggregate_body(src_hbm, dst_hbm, y_hbm, out_hbm, acc, *sc):
    rows = sc[:DR]
    sidx, didx = sc[DR], sc[DR + 1]
    isem = sc[DR + 2]
    gsem = sc[DR + 3:DR + 3 + DR]
    xsem = sc[DR + 3 + DR:]
    c = lax.axis_index("c")
    s = lax.axis_index("s")
    e0 = s * EPW
    coff = c * NP

    init = pltpu.make_async_copy(y_hbm.at[pl.ds(c * NP + s * RPS, RPS), :],
                                 acc.at[pl.ds(s * RPS, RPS), :], isem)
    init.start()

    def _idx_copies(j, q):
        base = e0 + j * CH
        return (pltpu.make_async_copy(src_hbm.at[pl.ds(base, CH)],
                                      sidx.at[q], xsem[q]),
                pltpu.make_async_copy(dst_hbm.at[pl.ds(base, CH)],
                                      didx.at[q], xsem[q]))

    def _fire_idx(j, q):
        for cp in _idx_copies(j, q):
            cp.start()

    def _gather(b, q):
        return pltpu.make_async_copy(y_hbm.at[sidx.at[q]], rows[b], gsem[b])

    def _prep(j, b, q):
        for cp in _idx_copies(j, q):
            cp.wait()
        for k in range(CH // 16):
            sidx[q, pl.ds(k * 16, 16)] = sidx[q, pl.ds(k * 16, 16)] + coff
        _gather(b, q).start()

    for q0 in range(6):
        _fire_idx(q0, q0)
    _prep(0, 0, 0)
    _prep(1, 1, 1)
    _prep(2, 2, 2)
    init.wait()
    plsc.subcore_barrier()

    def _outer(t, _):
        for u in range(DI):
            j = t * DI + u
            b, q = u % DR, u

            @pl.when(j < NCH)
            def _gw():
                _gather(b, q).wait()

            @pl.when(j + 3 < NCH)
            def _pp():
                _prep(j + 3, (u + 3) % DR, (u + 3) % DI)

            @pl.when(j < NCH)
            def _sc():
                pltpu.sync_copy(rows[b], acc.at[didx.at[q]], add=True)

            @pl.when(j + 6 < NCH)
            def _fi():
                _fire_idx(j + 6, (u + 6) % DI)
        return _
    lax.fori_loop(0, NST // DI, _outer, None)

    plsc.subcore_barrier()
    pltpu.sync_copy(acc.at[pl.ds(s * RPS, RPS), :],
                    out_hbm.at[c, pl.ds(s * RPS, RPS), :])


def _dinv_of(dp):
    deg = dp[0, :, 0] + dp[1, :, 0] + 1.0
    return lax.rsqrt(deg)[:, None]


def _tc_prologue_body(x_ref, w_ref, dp_ref, y_ref):
    dinv = _dinv_of(dp_ref[...])
    xw = jnp.dot(x_ref[...], w_ref[...], preferred_element_type=jnp.float32)
    y = xw * dinv
    y_ref[0] = y[:, :HH]
    y_ref[1] = y[:, HH:]


def _tc_prologue(x, W1, dp):
    return pl.pallas_call(
        _tc_prologue_body,
        grid=(NB,),
        in_specs=[
            pl.BlockSpec((R, IN), lambda i: (i, 0)),
            pl.BlockSpec((IN, HID), lambda i: (0, 0)),
            pl.BlockSpec((2, R, 16), lambda i: (0, i, 0)),
        ],
        out_specs=pl.BlockSpec((2, R, HH), lambda i: (0, i, 0)),
        out_shape=jax.ShapeDtypeStruct((2, NP, HH), jnp.float32),
    )(x, W1, dp)


def _tc_mid_body(a_ref, dp_ref, b_ref, w_ref, y_ref):
    dinv = _dinv_of(dp_ref[...])
    agg = jnp.concatenate([a_ref[0], a_ref[1]], axis=1)
    h = jax.nn.relu(agg * dinv + b_ref[...])
    hw = jnp.dot(h, w_ref[...], preferred_element_type=jnp.float32)
    y = hw * dinv
    y_ref[0] = y[:, :HH]
    y_ref[1] = y[:, HH:]


def _tc_mid(agg1, dp, b1, W2):
    return pl.pallas_call(
        _tc_mid_body,
        grid=(NB,),
        in_specs=[
            pl.BlockSpec((2, R, HH), lambda i: (0, i, 0)),
            pl.BlockSpec((2, R, 16), lambda i: (0, i, 0)),
            pl.BlockSpec((1, HID), lambda i: (0, 0)),
            pl.BlockSpec((HID, HID), lambda i: (0, 0)),
        ],
        out_specs=pl.BlockSpec((2, R, HH), lambda i: (0, i, 0)),
        out_shape=jax.ShapeDtypeStruct((2, NP, HH), jnp.float32),
    )(agg1, dp, b1, W2)


def _tc_final_body(a_ref, dp_ref, b_ref, batch_ref, wl_ref, bl_ref, o_ref,
                   pool_acc, cnt_acc):
    i = pl.program_id(0)
    dinv = _dinv_of(dp_ref[...])
    agg = jnp.concatenate([a_ref[0], a_ref[1]], axis=1)
    h = jax.nn.relu(agg * dinv + b_ref[...])

    bv = batch_ref[0, 0, :]
    sel = (lax.broadcasted_iota(jnp.int32, (G, R), 0) == bv[None, :])
    S = sel.astype(jnp.float32)

    @pl.when(i == 0)
    def _init():
        pool_acc[...] = jnp.zeros((G, HID), jnp.float32)
        cnt_acc[...] = jnp.zeros((G, 128), jnp.float32)

    pool_acc[...] += jnp.dot(S, h, preferred_element_type=jnp.float32)
    cnt_acc[...] += jnp.broadcast_to(jnp.sum(S, axis=1, keepdims=True),
                                     (G, 128))

    @pl.when(i == NB - 1)
    def _fin():
        cnt = cnt_acc[...][:, 0:1]
        pooled = pool_acc[...] / jnp.maximum(cnt, 1.0)
        o_ref[...] = (jnp.dot(pooled, wl_ref[...],
                              preferred_element_type=jnp.float32)
                      + bl_ref[...])


def _tc_final(agg2, dp, b2, batch3, Wl, bl):
    return pl.pallas_call(
        _tc_final_body,
        grid=(NB,),
        in_specs=[
            pl.BlockSpec((2, R, HH), lambda i: (0, i, 0)),
            pl.BlockSpec((2, R, 16), lambda i: (0, i, 0)),
            pl.BlockSpec((1, HID), lambda i: (0, 0)),
            pl.BlockSpec((1, 1, R), lambda i: (i, 0, 0)),
            pl.BlockSpec((HID, OUT), lambda i: (0, 0)),
            pl.BlockSpec((1, OUT), lambda i: (0, 0)),
        ],
        out_specs=pl.BlockSpec((G, OUT), lambda i: (0, 0)),
        out_shape=jax.ShapeDtypeStruct((G, OUT), jnp.float32),
        scratch_shapes=[
            pltpu.VMEM((G, HID), jnp.float32),
            pltpu.VMEM((G, 128), jnp.float32),
        ],
    )(agg2, dp, b2, batch3, Wl, bl)


def kernel(x, edge_index, batch, W1, b1, W2, b2, Wl, bl):
    src = edge_index[0]
    dst = edge_index[1]
    dp = _sc_degree_fn()(dst)
    y1 = _tc_prologue(x, W1, dp)
    agg1 = _sc_aggregate_fn()(src, dst, y1.reshape(2 * NP, HH))
    y2 = _tc_mid(agg1, dp, b1.reshape(1, HID), W2)
    agg2 = _sc_aggregate_fn()(src, dst, y2.reshape(2 * NP, HH))
    return _tc_final(agg2, dp, b2.reshape(1, HID),
                     batch.reshape(NB, 1, R), Wl, bl.reshape(1, OUT))


# speedup vs baseline: 1.7151x; 1.1312x over previous
import functools

import jax
import jax.numpy as jnp
from jax import lax
from jax.experimental import pallas as pl
from jax.experimental.pallas import tpu as pltpu
from jax.experimental.pallas import tpu_sc as plsc

N = 10000
E = 320000
IN = 128
HID = 256
HH = 128
OUT = 128
G = 64

NC = 2
NS = 16
CH = 80
NP = 10240
RPS = NP // NS

R = 400
NB = N // R

@functools.cache
def _sc_degree_fn():
    mesh = plsc.VectorSubcoreMesh(core_axis_name="c", subcore_axis_name="s")
    return functools.partial(
        pl.kernel,
        mesh=mesh,
        out_type=jax.ShapeDtypeStruct((NC, NP, 16), jnp.float32),
        scratch_types=[
            pltpu.VMEM_SHARED((NP, 16), jnp.float32),
            pltpu.VMEM((RPS, 16), jnp.float32),
            pltpu.VMEM((CH, 16), jnp.float32),
            pltpu.VMEM((E // (NC * NS),), jnp.int32),
            pltpu.VMEM((E // (NC * NS * CH), CH), jnp.int32),
            pltpu.SemaphoreType.DMA,
            pltpu.SemaphoreType.DMA,
        ],
    )(_sc_degree_body)


def _sc_degree_body(dst_hbm, out_hbm, acc, zbuf, obuf, dtmp, didx2, tsem,
                    ssem):
    c = lax.axis_index("c")
    s = lax.axis_index("s")
    epw = E // (NC * NS)
    nch = epw // CH
    base0 = (c * NS + s) * epw

    pltpu.make_async_copy(dst_hbm.at[pl.ds(base0, epw)], dtmp, tsem).start()

    def _zrow(i, _):
        zbuf[i, :] = jnp.zeros((16,), jnp.float32)
        return _
    lax.fori_loop(0, RPS, _zrow, None)

    def _orow(i, _):
        obuf[i, :] = jnp.full((16,), 1.0, jnp.float32)
        return _
    lax.fori_loop(0, CH, _orow, None)

    pltpu.sync_copy(zbuf, acc.at[pl.ds(s * RPS, RPS), :])

    pltpu.make_async_copy(dst_hbm.at[pl.ds(base0, epw)], dtmp, tsem).wait()

    def _dunp(i, _):
        j = i // (CH // 16)
        k = i - j * (CH // 16)
        didx2[j, pl.ds(k * 16, 16)] = dtmp[pl.ds(i * 16, 16)]
        return _
    lax.fori_loop(0, epw // 16, _dunp, None)

    plsc.subcore_barrier()

    W = 16

    def _scat(j):
        return pltpu.make_async_copy(obuf, acc.at[didx2.at[j]], ssem)

    def _fire(j, _):
        _scat(j).start(add=True)

        @pl.when(j >= W)
        def _drain():
            _scat(j - W).wait()
        return _
    lax.fori_loop(0, nch, _fire, None)

    def _tail(j, _):
        _scat(j).wait()
        return _
    lax.fori_loop(nch - W, nch, _tail, None)

    plsc.subcore_barrier()
    pltpu.sync_copy(acc.at[pl.ds(s * RPS, RPS), :],
                    out_hbm.at[c, pl.ds(s * RPS, RPS), :])


EPW = E // NS
NCH = EPW // CH
DR = 4
DI = 8
NST = 256


@functools.cache
def _sc_aggregate_fn():
    mesh = plsc.VectorSubcoreMesh(core_axis_name="c", subcore_axis_name="s")
    return functools.partial(
        pl.kernel,
        mesh=mesh,
        out_type=jax.ShapeDtypeStruct((NC, NP, HH), jnp.float32),
        scratch_types=(
            [pltpu.VMEM_SHARED((NP, HH), jnp.float32)]
            + [pltpu.VMEM((CH, HH), jnp.float32) for _ in range(DR)]
            + [pltpu.VMEM((DI, CH), jnp.int32),
               pltpu.VMEM((DI, CH), jnp.int32)]
            + [pltpu.SemaphoreType.DMA for _ in range(1 + DR + DI)]
        ),
    )(_sc_aggregate_body)


def _sc_aggregate_body(src_hbm, dst_hbm, y_hbm, out_hbm, acc, *sc):
    rows = sc[:DR]
    sidx, didx = sc[DR], sc[DR + 1]
    isem = sc[DR + 2]
    gsem = sc[DR + 3:DR + 3 + DR]
    xsem = sc[DR + 3 + DR:]
    c = lax.axis_index("c")
    s = lax.axis_index("s")
    e0 = s * EPW
    coff = c * NP

    init = pltpu.make_async_copy(y_hbm.at[pl.ds(c * NP + s * RPS, RPS), :],
                                 acc.at[pl.ds(s * RPS, RPS), :], isem)
    init.start()

    def _idx_copies(j, q):
        base = e0 + j * CH
        return (pltpu.make_async_copy(src_hbm.at[pl.ds(base, CH)],
                                      sidx.at[q], xsem[q]),
                pltpu.make_async_copy(dst_hbm.at[pl.ds(base, CH)],
                                      didx.at[q], xsem[q]))

    def _fire_idx(j, q):
        for cp in _idx_copies(j, q):
            cp.start()

    def _gather(b, q):
        return pltpu.make_async_copy(y_hbm.at[sidx.at[q]], rows[b], gsem[b])

    def _prep(j, b, q):
        for cp in _idx_copies(j, q):
            cp.wait()
        for k in range(CH // 16):
            sidx[q, pl.ds(k * 16, 16)] = sidx[q, pl.ds(k * 16, 16)] + coff
        _gather(b, q).start()

    for q0 in range(6):
        _fire_idx(q0, q0)
    _prep(0, 0, 0)
    _prep(1, 1, 1)
    _prep(2, 2, 2)
    init.wait()
    plsc.subcore_barrier()

    def _outer(t, _):
        for u in range(DI):
            j = t * DI + u
            b, q = u % DR, u

            @pl.when(j < NCH)
            def _gw():
                _gather(b, q).wait()

            @pl.when(j + 3 < NCH)
            def _pp():
                _prep(j + 3, (u + 3) % DR, (u + 3) % DI)

            @pl.when(j < NCH)
            def _sc():
                pltpu.sync_copy(rows[b], acc.at[didx.at[q]], add=True)

            @pl.when(j + 6 < NCH)
            def _fi():
                _fire_idx(j + 6, (u + 6) % DI)
        return _
    lax.fori_loop(0, NST // DI, _outer, None)

    plsc.subcore_barrier()
    pltpu.sync_copy(acc.at[pl.ds(s * RPS, RPS), :],
                    out_hbm.at[c, pl.ds(s * RPS, RPS), :])


def _dinv_of(dp):
    deg = dp[0, :, 0] + dp[1, :, 0] + 1.0
    return lax.rsqrt(deg)[:, None]


def _tc_prologue_body(x_ref, w_ref, dp_ref, y_ref):
    dinv = _dinv_of(dp_ref[...])
    xw = jnp.dot(x_ref[...], w_ref[...], preferred_element_type=jnp.float32)
    y = xw * dinv
    y_ref[0] = y[:, :HH]
    y_ref[1] = y[:, HH:]


def _tc_prologue(x, W1, dp):
    return pl.pallas_call(
        _tc_prologue_body,
        grid=(NB,),
        in_specs=[
            pl.BlockSpec((R, IN), lambda i: (i, 0)),
            pl.BlockSpec((IN, HID), lambda i: (0, 0)),
            pl.BlockSpec((2, R, 16), lambda i: (0, i, 0)),
        ],
        out_specs=pl.BlockSpec((2, R, HH), lambda i: (0, i, 0)),
        out_shape=jax.ShapeDtypeStruct((2, NP, HH), jnp.float32),
    )(x, W1, dp)


def _tc_mid_body(a_ref, dp_ref, b_ref, w_ref, y_ref):
    dinv = _dinv_of(dp_ref[...])
    agg = jnp.concatenate([a_ref[0], a_ref[1]], axis=1)
    h = jax.nn.relu(agg * dinv + b_ref[...])
    hw = jnp.dot(h, w_ref[...], preferred_element_type=jnp.float32)
    y = hw * dinv
    y_ref[0] = y[:, :HH]
    y_ref[1] = y[:, HH:]


def _tc_mid(agg1, dp, b1, W2):
    return pl.pallas_call(
        _tc_mid_body,
        grid=(NB,),
        in_specs=[
            pl.BlockSpec((2, R, HH), lambda i: (0, i, 0)),
            pl.BlockSpec((2, R, 16), lambda i: (0, i, 0)),
            pl.BlockSpec((1, HID), lambda i: (0, 0)),
            pl.BlockSpec((HID, HID), lambda i: (0, 0)),
        ],
        out_specs=pl.BlockSpec((2, R, HH), lambda i: (0, i, 0)),
        out_shape=jax.ShapeDtypeStruct((2, NP, HH), jnp.float32),
    )(agg1, dp, b1, W2)


def _tc_final_body(a_ref, dp_ref, b_ref, batch_ref, wl_ref, bl_ref, o_ref,
                   pool_acc, cnt_acc):
    i = pl.program_id(0)
    dinv = _dinv_of(dp_ref[...])
    agg = jnp.concatenate([a_ref[0], a_ref[1]], axis=1)
    h = jax.nn.relu(agg * dinv + b_ref[...])

    bv = batch_ref[0, 0, :]
    sel = (lax.broadcasted_iota(jnp.int32, (G, R), 0) == bv[None, :])
    S = sel.astype(jnp.float32)

    @pl.when(i == 0)
    def _init():
        pool_acc[...] = jnp.zeros((G, HID), jnp.float32)
        cnt_acc[...] = jnp.zeros((G, 128), jnp.float32)

    pool_acc[...] += jnp.dot(S, h, preferred_element_type=jnp.float32)
    cnt_acc[...] += jnp.broadcast_to(jnp.sum(S, axis=1, keepdims=True),
                                     (G, 128))

    @pl.when(i == NB - 1)
    def _fin():
        cnt = cnt_acc[...][:, 0:1]
        pooled = pool_acc[...] / jnp.maximum(cnt, 1.0)
        o_ref[...] = (jnp.dot(pooled, wl_ref[...],
                              preferred_element_type=jnp.float32)
                      + bl_ref[...])


def _tc_final(agg2, dp, b2, batch3, Wl, bl):
    return pl.pallas_call(
        _tc_final_body,
        grid=(NB,),
        in_specs=[
            pl.BlockSpec((2, R, HH), lambda i: (0, i, 0)),
            pl.BlockSpec((2, R, 16), lambda i: (0, i, 0)),
            pl.BlockSpec((1, HID), lambda i: (0, 0)),
            pl.BlockSpec((1, 1, R), lambda i: (i, 0, 0)),
            pl.BlockSpec((HID, OUT), lambda i: (0, 0)),
            pl.BlockSpec((1, OUT), lambda i: (0, 0)),
        ],
        out_specs=pl.BlockSpec((G, OUT), lambda i: (0, 0)),
        out_shape=jax.ShapeDtypeStruct((G, OUT), jnp.float32),
        scratch_shapes=[
            pltpu.VMEM((G, HID), jnp.float32),
            pltpu.VMEM((G, 128), jnp.float32),
        ],
    )(agg2, dp, b2, batch3, Wl, bl)


def kernel(x, edge_index, batch, W1, b1, W2, b2, Wl, bl):
    src = edge_index[0]
    dst = edge_index[1]
    dp = _sc_degree_fn()(dst)
    y1 = _tc_prologue(x, W1, dp)
    agg1 = _sc_aggregate_fn()(src, dst, y1.reshape(2 * NP, HH))
    y2 = _tc_mid(agg1, dp, b1.reshape(1, HID), W2)
    agg2 = _sc_aggregate_fn()(src, dst, y2.reshape(2 * NP, HH))
    return _tc_final(agg2, dp, b2.reshape(1, HID),
                     batch.reshape(NB, 1, R), Wl, bl.reshape(1, OUT))


# speedup vs baseline: 1.7154x; 1.0002x over previous
import functools

import jax
import jax.numpy as jnp
from jax import lax
from jax.experimental import pallas as pl
from jax.experimental.pallas import tpu as pltpu
from jax.experimental.pallas import tpu_sc as plsc

N = 10000
E = 320000
IN = 128
HID = 256
HH = 128
OUT = 128
G = 64

NC = 2
NS = 16
CH = 80
NP = 10240
RPS = NP // NS

R = 400
NB = N // R

@functools.cache
def _sc_degree_fn():
    mesh = plsc.VectorSubcoreMesh(core_axis_name="c", subcore_axis_name="s")
    return functools.partial(
        pl.kernel,
        mesh=mesh,
        out_type=jax.ShapeDtypeStruct((NC, NP, 16), jnp.float32),
        scratch_types=[
            pltpu.VMEM_SHARED((NP, 16), jnp.float32),
            pltpu.VMEM((RPS, 16), jnp.float32),
            pltpu.VMEM((CH, 16), jnp.float32),
            pltpu.VMEM((E // (NC * NS),), jnp.int32),
            pltpu.VMEM((E // (NC * NS * CH), CH), jnp.int32),
            pltpu.SemaphoreType.DMA,
            pltpu.SemaphoreType.DMA,
        ],
    )(_sc_degree_body)


def _sc_degree_body(dst_hbm, out_hbm, acc, zbuf, obuf, dtmp, didx2, tsem,
                    ssem):
    c = lax.axis_index("c")
    s = lax.axis_index("s")
    epw = E // (NC * NS)
    nch = epw // CH
    base0 = (c * NS + s) * epw

    pltpu.make_async_copy(dst_hbm.at[pl.ds(base0, epw)], dtmp, tsem).start()

    def _zrow(i, _):
        zbuf[i, :] = jnp.zeros((16,), jnp.float32)
        return _
    lax.fori_loop(0, RPS, _zrow, None)

    def _orow(i, _):
        obuf[i, :] = jnp.full((16,), 1.0, jnp.float32)
        return _
    lax.fori_loop(0, CH, _orow, None)

    pltpu.sync_copy(zbuf, acc.at[pl.ds(s * RPS, RPS), :])

    pltpu.make_async_copy(dst_hbm.at[pl.ds(base0, epw)], dtmp, tsem).wait()

    def _dunp(i, _):
        j = i // (CH // 16)
        k = i - j * (CH // 16)
        didx2[j, pl.ds(k * 16, 16)] = dtmp[pl.ds(i * 16, 16)]
        return _
    lax.fori_loop(0, epw // 16, _dunp, None)

    plsc.subcore_barrier()

    W = 16

    def _scat(j):
        return pltpu.make_async_copy(obuf, acc.at[didx2.at[j]], ssem)

    def _fire(j, _):
        _scat(j).start(add=True)

        @pl.when(j >= W)
        def _drain():
            _scat(j - W).wait()
        return _
    lax.fori_loop(0, nch, _fire, None)

    def _tail(j, _):
        _scat(j).wait()
        return _
    lax.fori_loop(nch - W, nch, _tail, None)

    plsc.subcore_barrier()
    pltpu.sync_copy(acc.at[pl.ds(s * RPS, RPS), :],
                    out_hbm.at[c, pl.ds(s * RPS, RPS), :])


EPW = E // NS
NCH = EPW // CH
DR = 4
DI = 8
NST = 256


@functools.cache
def _sc_aggregate_fn():
    mesh = plsc.VectorSubcoreMesh(core_axis_name="c", subcore_axis_name="s")
    return functools.partial(
        pl.kernel,
        mesh=mesh,
        out_type=jax.ShapeDtypeStruct((NC, NP, HH), jnp.float32),
        scratch_types=(
            [pltpu.VMEM_SHARED((NP, HH), jnp.float32)]
            + [pltpu.VMEM((CH, HH), jnp.float32) for _ in range(DR)]
            + [pltpu.VMEM((DI, CH), jnp.int32),
               pltpu.VMEM((DI, CH), jnp.int32)]
            + [pltpu.SemaphoreType.DMA for _ in range(1 + 2 * DR + DI)]
        ),
    )(_sc_aggregate_body)


def _sc_aggregate_body(src_hbm, dst_hbm, y_hbm, out_hbm, acc, *sc):
    rows = sc[:DR]
    sidx, didx = sc[DR], sc[DR + 1]
    isem = sc[DR + 2]
    gsem = sc[DR + 3:DR + 3 + DR]
    ssem = sc[DR + 3 + DR:DR + 3 + 2 * DR]
    xsem = sc[DR + 3 + 2 * DR:]
    c = lax.axis_index("c")
    s = lax.axis_index("s")
    e0 = s * EPW
    coff = c * NP

    init = pltpu.make_async_copy(y_hbm.at[pl.ds(c * NP + s * RPS, RPS), :],
                                 acc.at[pl.ds(s * RPS, RPS), :], isem)
    init.start()

    def _idx_copies(j, q):
        base = e0 + j * CH
        return (pltpu.make_async_copy(src_hbm.at[pl.ds(base, CH)],
                                      sidx.at[q], xsem[q]),
                pltpu.make_async_copy(dst_hbm.at[pl.ds(base, CH)],
                                      didx.at[q], xsem[q]))

    def _fire_idx(j, q):
        for cp in _idx_copies(j, q):
            cp.start()

    def _gather(b, q):
        return pltpu.make_async_copy(y_hbm.at[sidx.at[q]], rows[b], gsem[b])

    def _prep(j, b, q):
        for cp in _idx_copies(j, q):
            cp.wait()
        for k in range(CH // 16):
            sidx[q, pl.ds(k * 16, 16)] = sidx[q, pl.ds(k * 16, 16)] + coff
        _gather(b, q).start()

    for q0 in range(6):
        _fire_idx(q0, q0)
    _prep(0, 0, 0)
    _prep(1, 1, 1)
    _prep(2, 2, 2)
    init.wait()
    plsc.subcore_barrier()

    def _scatter(b, q):
        return pltpu.make_async_copy(rows[b], acc.at[didx.at[q]], ssem[b])

    def _outer(t, _):
        for u in range(DI):
            j = t * DI + u
            b, q = u % DR, u

            @pl.when(j < NCH)
            def _gw():
                _gather(b, q).wait()

            @pl.when((j >= 1) & (j < NCH + 1))
            def _sw():
                _scatter((u + 3) % DR, (u + 7) % DI).wait()

            @pl.when(j + 3 < NCH)
            def _pp():
                _prep(j + 3, (u + 3) % DR, (u + 3) % DI)

            @pl.when(j < NCH)
            def _sc():
                _scatter(b, q).start(add=True)

            @pl.when(j + 6 < NCH)
            def _fi():
                _fire_idx(j + 6, (u + 6) % DI)
        return _
    lax.fori_loop(0, NST // DI, _outer, None)

    plsc.subcore_barrier()
    pltpu.sync_copy(acc.at[pl.ds(s * RPS, RPS), :],
                    out_hbm.at[c, pl.ds(s * RPS, RPS), :])


def _dinv_of(dp):
    deg = dp[0, :, 0] + dp[1, :, 0] + 1.0
    return lax.rsqrt(deg)[:, None]


def _tc_prologue_body(x_ref, w_ref, dp_ref, y_ref):
    dinv = _dinv_of(dp_ref[...])
    xw = jnp.dot(x_ref[...], w_ref[...], preferred_element_type=jnp.float32)
    y = xw * dinv
    y_ref[0] = y[:, :HH]
    y_ref[1] = y[:, HH:]


def _tc_prologue(x, W1, dp):
    return pl.pallas_call(
        _tc_prologue_body,
        grid=(NB,),
        in_specs=[
            pl.BlockSpec((R, IN), lambda i: (i, 0)),
            pl.BlockSpec((IN, HID), lambda i: (0, 0)),
            pl.BlockSpec((2, R, 16), lambda i: (0, i, 0)),
        ],
        out_specs=pl.BlockSpec((2, R, HH), lambda i: (0, i, 0)),
        out_shape=jax.ShapeDtypeStruct((2, NP, HH), jnp.float32),
    )(x, W1, dp)


def _tc_mid_body(a_ref, dp_ref, b_ref, w_ref, y_ref):
    dinv = _dinv_of(dp_ref[...])
    agg = jnp.concatenate([a_ref[0], a_ref[1]], axis=1)
    h = jax.nn.relu(agg * dinv + b_ref[...])
    hw = jnp.dot(h, w_ref[...], preferred_element_type=jnp.float32)
    y = hw * dinv
    y_ref[0] = y[:, :HH]
    y_ref[1] = y[:, HH:]


def _tc_mid(agg1, dp, b1, W2):
    return pl.pallas_call(
        _tc_mid_body,
        grid=(NB,),
        in_specs=[
            pl.BlockSpec((2, R, HH), lambda i: (0, i, 0)),
            pl.BlockSpec((2, R, 16), lambda i: (0, i, 0)),
            pl.BlockSpec((1, HID), lambda i: (0, 0)),
            pl.BlockSpec((HID, HID), lambda i: (0, 0)),
        ],
        out_specs=pl.BlockSpec((2, R, HH), lambda i: (0, i, 0)),
        out_shape=jax.ShapeDtypeStruct((2, NP, HH), jnp.float32),
    )(agg1, dp, b1, W2)


def _tc_final_body(a_ref, dp_ref, b_ref, batch_ref, wl_ref, bl_ref, o_ref,
                   pool_acc, cnt_acc):
    i = pl.program_id(0)
    dinv = _dinv_of(dp_ref[...])
    agg = jnp.concatenate([a_ref[0], a_ref[1]], axis=1)
    h = jax.nn.relu(agg * dinv + b_ref[...])

    bv = batch_ref[0, 0, :]
    sel = (lax.broadcasted_iota(jnp.int32, (G, R), 0) == bv[None, :])
    S = sel.astype(jnp.float32)

    @pl.when(i == 0)
    def _init():
        pool_acc[...] = jnp.zeros((G, HID), jnp.float32)
        cnt_acc[...] = jnp.zeros((G, 128), jnp.float32)

    pool_acc[...] += jnp.dot(S, h, preferred_element_type=jnp.float32)
    cnt_acc[...] += jnp.broadcast_to(jnp.sum(S, axis=1, keepdims=True),
                                     (G, 128))

    @pl.when(i == NB - 1)
    def _fin():
        cnt = cnt_acc[...][:, 0:1]
        pooled = pool_acc[...] / jnp.maximum(cnt, 1.0)
        o_ref[...] = (jnp.dot(pooled, wl_ref[...],
                              preferred_element_type=jnp.float32)
                      + bl_ref[...])


def _tc_final(agg2, dp, b2, batch3, Wl, bl):
    return pl.pallas_call(
        _tc_final_body,
        grid=(NB,),
        in_specs=[
            pl.BlockSpec((2, R, HH), lambda i: (0, i, 0)),
            pl.BlockSpec((2, R, 16), lambda i: (0, i, 0)),
            pl.BlockSpec((1, HID), lambda i: (0, 0)),
            pl.BlockSpec((1, 1, R), lambda i: (i, 0, 0)),
            pl.BlockSpec((HID, OUT), lambda i: (0, 0)),
            pl.BlockSpec((1, OUT), lambda i: (0, 0)),
        ],
        out_specs=pl.BlockSpec((G, OUT), lambda i: (0, 0)),
        out_shape=jax.ShapeDtypeStruct((G, OUT), jnp.float32),
        scratch_shapes=[
            pltpu.VMEM((G, HID), jnp.float32),
            pltpu.VMEM((G, 128), jnp.float32),
        ],
    )(agg2, dp, b2, batch3, Wl, bl)


def kernel(x, edge_index, batch, W1, b1, W2, b2, Wl, bl):
    src = edge_index[0]
    dst = edge_index[1]
    dp = _sc_degree_fn()(dst)
    y1 = _tc_prologue(x, W1, dp)
    agg1 = _sc_aggregate_fn()(src, dst, y1.reshape(2 * NP, HH))
    y2 = _tc_mid(agg1, dp, b1.reshape(1, HID), W2)
    agg2 = _sc_aggregate_fn()(src, dst, y2.reshape(2 * NP, HH))
    return _tc_final(agg2, dp, b2.reshape(1, HID),
                     batch.reshape(NB, 1, R), Wl, bl.reshape(1, OUT))


# speedup vs baseline: 1.7168x; 1.0008x over previous
import functools

import jax
import jax.numpy as jnp
from jax import lax
from jax.experimental import pallas as pl
from jax.experimental.pallas import tpu as pltpu
from jax.experimental.pallas import tpu_sc as plsc

N = 10000
E = 320000
IN = 128
HID = 256
HH = 128
OUT = 128
G = 64

NC = 2
NS = 16
CH = 80
NP = 10240
RPS = NP // NS

R = 400
NB = N // R

@functools.cache
def _sc_degree_fn():
    mesh = plsc.VectorSubcoreMesh(core_axis_name="c", subcore_axis_name="s")
    return functools.partial(
        pl.kernel,
        mesh=mesh,
        out_type=jax.ShapeDtypeStruct((NC, NP, 16), jnp.float32),
        scratch_types=[
            pltpu.VMEM_SHARED((NP, 16), jnp.float32),
            pltpu.VMEM((RPS, 16), jnp.float32),
            pltpu.VMEM((CH, 16), jnp.float32),
            pltpu.VMEM((E // (NC * NS),), jnp.int32),
            pltpu.VMEM((E // (NC * NS * CH), CH), jnp.int32),
            pltpu.SemaphoreType.DMA,
            pltpu.SemaphoreType.DMA,
        ],
    )(_sc_degree_body)


def _sc_degree_body(dst_hbm, out_hbm, acc, zbuf, obuf, dtmp, didx2, tsem,
                    ssem):
    c = lax.axis_index("c")
    s = lax.axis_index("s")
    epw = E // (NC * NS)
    nch = epw // CH
    base0 = (c * NS + s) * epw

    pltpu.make_async_copy(dst_hbm.at[pl.ds(base0, epw)], dtmp, tsem).start()

    def _zrow(i, _):
        zbuf[i, :] = jnp.zeros((16,), jnp.float32)
        return _
    lax.fori_loop(0, RPS, _zrow, None)

    def _orow(i, _):
        obuf[i, :] = jnp.full((16,), 1.0, jnp.float32)
        return _
    lax.fori_loop(0, CH, _orow, None)

    pltpu.sync_copy(zbuf, acc.at[pl.ds(s * RPS, RPS), :])

    pltpu.make_async_copy(dst_hbm.at[pl.ds(base0, epw)], dtmp, tsem).wait()

    def _dunp(i, _):
        j = i // (CH // 16)
        k = i - j * (CH // 16)
        didx2[j, pl.ds(k * 16, 16)] = dtmp[pl.ds(i * 16, 16)]
        return _
    lax.fori_loop(0, epw // 16, _dunp, None)

    plsc.subcore_barrier()

    W = 16

    def _scat(j):
        return pltpu.make_async_copy(obuf, acc.at[didx2.at[j]], ssem)

    def _fire(j, _):
        _scat(j).start(add=True)

        @pl.when(j >= W)
        def _drain():
            _scat(j - W).wait()
        return _
    lax.fori_loop(0, nch, _fire, None)

    def _tail(j, _):
        _scat(j).wait()
        return _
    lax.fori_loop(nch - W, nch, _tail, None)

    plsc.subcore_barrier()
    pltpu.sync_copy(acc.at[pl.ds(s * RPS, RPS), :],
                    out_hbm.at[c, pl.ds(s * RPS, RPS), :])


EPW = E // NS
NCH = EPW // CH
DR = 4
DI = 8
NST = 256


@functools.cache
def _sc_aggregate_fn():
    mesh = plsc.VectorSubcoreMesh(core_axis_name="c", subcore_axis_name="s")
    return functools.partial(
        pl.kernel,
        mesh=mesh,
        out_type=jax.ShapeDtypeStruct((NC, NP, HH), jnp.float32),
        scratch_types=(
            [pltpu.VMEM_SHARED((NP, HH), jnp.float32)]
            + [pltpu.VMEM((CH, HH), jnp.float32) for _ in range(DR)]
            + [pltpu.VMEM((DI, CH), jnp.int32),
               pltpu.VMEM((DI, CH), jnp.int32)]
            + [pltpu.SemaphoreType.DMA for _ in range(1 + 2 * DR + DI)]
        ),
    )(_sc_aggregate_body)


def _sc_aggregate_body(src_hbm, dst_hbm, y_hbm, out_hbm, acc, *sc):
    rows = sc[:DR]
    sidx, didx = sc[DR], sc[DR + 1]
    isem = sc[DR + 2]
    gsem = sc[DR + 3:DR + 3 + DR]
    ssem = sc[DR + 3 + DR:DR + 3 + 2 * DR]
    xsem = sc[DR + 3 + 2 * DR:]
    c = lax.axis_index("c")
    s = lax.axis_index("s")
    e0 = s * EPW
    coff = c * NP

    init = pltpu.make_async_copy(y_hbm.at[pl.ds(c * NP + s * RPS, RPS), :],
                                 acc.at[pl.ds(s * RPS, RPS), :], isem)
    init.start()

    def _idx_copies(j, q):
        base = e0 + j * CH
        return (pltpu.make_async_copy(src_hbm.at[pl.ds(base, CH)],
                                      sidx.at[q], xsem[q]),
                pltpu.make_async_copy(dst_hbm.at[pl.ds(base, CH)],
                                      didx.at[q], xsem[q]))

    def _fire_idx(j, q):
        for cp in _idx_copies(j, q):
            cp.start()

    def _gather(b, q):
        return pltpu.make_async_copy(y_hbm.at[sidx.at[q]], rows[b], gsem[b])

    def _prep(j, b, q):
        for cp in _idx_copies(j, q):
            cp.wait()
        for k in range(CH // 16):
            sidx[q, pl.ds(k * 16, 16)] = sidx[q, pl.ds(k * 16, 16)] + coff
        _gather(b, q).start()

    for q0 in range(6):
        _fire_idx(q0, q0)
    _prep(0, 0, 0)
    _prep(1, 1, 1)
    _prep(2, 2, 2)
    init.wait()
    plsc.subcore_barrier()

    def _outer(t, _):
        for u in range(DI):
            j = t * DI + u
            b, q = u % DR, u

            @pl.when(j < NCH)
            def _gw():
                _gather(b, q).wait()

            @pl.when(j + 3 < NCH)
            def _pp():
                _prep(j + 3, (u + 3) % DR, (u + 3) % DI)

            @pl.when(j < NCH)
            def _sc():
                pltpu.sync_copy(rows[b], acc.at[didx.at[q]], add=True)

            @pl.when(j + 6 < NCH)
            def _fi():
                _fire_idx(j + 6, (u + 6) % DI)
        return _
    lax.fori_loop(0, NST // DI, _outer, None)

    plsc.subcore_barrier()
    pltpu.sync_copy(acc.at[pl.ds(s * RPS, RPS), :],
                    out_hbm.at[c, pl.ds(s * RPS, RPS), :])


def _dinv_of(dp):
    deg = dp[0, :, 0] + dp[1, :, 0] + 1.0
    return lax.rsqrt(deg)[:, None]


def _tc_prologue_body(x_ref, w_ref, dp_ref, y_ref):
    dinv = _dinv_of(dp_ref[...])
    xw = jnp.dot(x_ref[...], w_ref[...], preferred_element_type=jnp.float32)
    y = xw * dinv
    y_ref[0] = y[:, :HH]
    y_ref[1] = y[:, HH:]


def _tc_prologue(x, W1, dp):
    return pl.pallas_call(
        _tc_prologue_body,
        grid=(NB,),
        in_specs=[
            pl.BlockSpec((R, IN), lambda i: (i, 0)),
            pl.BlockSpec((IN, HID), lambda i: (0, 0)),
            pl.BlockSpec((2, R, 16), lambda i: (0, i, 0)),
        ],
        out_specs=pl.BlockSpec((2, R, HH), lambda i: (0, i, 0)),
        out_shape=jax.ShapeDtypeStruct((2, NP, HH), jnp.float32),
    )(x, W1, dp)


def _tc_mid_body(a_ref, dp_ref, b_ref, w_ref, y_ref):
    dinv = _dinv_of(dp_ref[...])
    agg = jnp.concatenate([a_ref[0], a_ref[1]], axis=1)
    h = jax.nn.relu(agg * dinv + b_ref[...])
    hw = jnp.dot(h, w_ref[...], preferred_element_type=jnp.float32)
    y = hw * dinv
    y_ref[0] = y[:, :HH]
    y_ref[1] = y[:, HH:]


def _tc_mid(agg1, dp, b1, W2):
    return pl.pallas_call(
        _tc_mid_body,
        grid=(NB,),
        in_specs=[
            pl.BlockSpec((2, R, HH), lambda i: (0, i, 0)),
            pl.BlockSpec((2, R, 16), lambda i: (0, i, 0)),
            pl.BlockSpec((1, HID), lambda i: (0, 0)),
            pl.BlockSpec((HID, HID), lambda i: (0, 0)),
        ],
        out_specs=pl.BlockSpec((2, R, HH), lambda i: (0, i, 0)),
        out_shape=jax.ShapeDtypeStruct((2, NP, HH), jnp.float32),
    )(agg1, dp, b1, W2)


def _tc_final_body(a_ref, dp_ref, b_ref, batch_ref, wl_ref, bl_ref, o_ref,
                   pool_acc, cnt_acc):
    i = pl.program_id(0)
    dinv = _dinv_of(dp_ref[...])
    agg = jnp.concatenate([a_ref[0], a_ref[1]], axis=1)
    h = jax.nn.relu(agg * dinv + b_ref[...])

    bv = batch_ref[0, 0, :]
    sel = (lax.broadcasted_iota(jnp.int32, (G, R), 0) == bv[None, :])
    S = sel.astype(jnp.float32)

    @pl.when(i == 0)
    def _init():
        pool_acc[...] = jnp.zeros((G, HID), jnp.float32)
        cnt_acc[...] = jnp.zeros((G, 128), jnp.float32)

    pool_acc[...] += jnp.dot(S, h, preferred_element_type=jnp.float32)
    cnt_acc[...] += jnp.broadcast_to(jnp.sum(S, axis=1, keepdims=True),
                                     (G, 128))

    @pl.when(i == NB - 1)
    def _fin():
        cnt = cnt_acc[...][:, 0:1]
        pooled = pool_acc[...] / jnp.maximum(cnt, 1.0)
        o_ref[...] = (jnp.dot(pooled, wl_ref[...],
                              preferred_element_type=jnp.float32)
                      + bl_ref[...])


def _tc_final(agg2, dp, b2, batch3, Wl, bl):
    return pl.pallas_call(
        _tc_final_body,
        grid=(NB,),
        in_specs=[
            pl.BlockSpec((2, R, HH), lambda i: (0, i, 0)),
            pl.BlockSpec((2, R, 16), lambda i: (0, i, 0)),
            pl.BlockSpec((1, HID), lambda i: (0, 0)),
            pl.BlockSpec((1, 1, R), lambda i: (i, 0, 0)),
            pl.BlockSpec((HID, OUT), lambda i: (0, 0)),
            pl.BlockSpec((1, OUT), lambda i: (0, 0)),
        ],
        out_specs=pl.BlockSpec((G, OUT), lambda i: (0, 0)),
        out_shape=jax.ShapeDtypeStruct((G, OUT), jnp.float32),
        scratch_shapes=[
            pltpu.VMEM((G, HID), jnp.float32),
            pltpu.VMEM((G, 128), jnp.float32),
        ],
    )(agg2, dp, b2, batch3, Wl, bl)


def kernel(x, edge_index, batch, W1, b1, W2, b2, Wl, bl):
    src = edge_index[0]
    dst = edge_index[1]
    dp = _sc_degree_fn()(dst)
    y1 = _tc_prologue(x, W1, dp)
    agg1 = _sc_aggregate_fn()(src, dst, y1.reshape(2 * NP, HH))
    y2 = _tc_mid(agg1, dp, b1.reshape(1, HID), W2)
    agg2 = _sc_aggregate_fn()(src, dst, y2.reshape(2 * NP, HH))
    return _tc_final(agg2, dp, b2.reshape(1, HID),
                     batch.reshape(NB, 1, R), Wl, bl.reshape(1, OUT))
